# baseline (speedup 1.0000x reference)
import functools

import jax
import jax.numpy as jnp
from jax import lax
from jax.experimental import pallas as pl
from jax.experimental.pallas import tpu as pltpu
from jax.experimental.pallas import tpu_sc as plsc

N_ATOM = 50000
N_EDGE = 50000
D = 128
H = 64
OUT = 10
NTAB = 5
NSTREAM = 9

NC = 2
NS = 16
NW = NC * NS
CHUNK = 56
CHUNKS_PER_TILE = 28
PER_TILE = CHUNK * CHUNKS_PER_TILE
NPAD = PER_TILE * NW

ATOM_BLK = 2000
ROW_BLK = 1568


def _proj_body(x_ref, w_ref, out_ref):
    y = jnp.dot(x_ref[...], w_ref[...], preferred_element_type=jnp.float32)
    for k in range(NTAB):
        out_ref[k] = y[:, k * D:(k + 1) * D]


def _project(x_atom, wcat):
    return pl.pallas_call(
        _proj_body,
        grid=(N_ATOM // ATOM_BLK,),
        in_specs=[
            pl.BlockSpec((ATOM_BLK, D), lambda i: (i, 0)),
            pl.BlockSpec((D, NTAB * D), lambda i: (0, 0)),
        ],
        out_specs=pl.BlockSpec((NTAB, ATOM_BLK, D), lambda i: (0, i, 0)),
        out_shape=jax.ShapeDtypeStruct((NTAB, N_ATOM, D), jnp.float32),
    )(x_atom, wcat)


def _sc_body(table, idx, out,
             i0, i1, i2, i3, i4, i5, i6, i7, i8,
             gb0, gb1, ga0, ga1, ga2, gt0, gt1, gt2, gt3,
             ob, oa, ot,
             sem_i, sem_b, sem_a, sem_t, sem_ob, sem_oa, sem_ot):
    ib = (i0, i1, i2, i3, i4, i5, i6, i7, i8)
    wid = lax.axis_index("s") * NC + lax.axis_index("c")
    base = wid * PER_TILE

    cps = [pltpu.async_copy(idx.at[pl.ds(k * NPAD + base, PER_TILE)], ib[k], sem_i)
           for k in range(NSTREAM)]
    for cp in cps:
        cp.wait()

    def fire_bond(off):
        pltpu.async_copy(table.at[i0.at[off]], gb0, sem_b)
        pltpu.async_copy(table.at[i1.at[off]], gb1, sem_b)

    def fire_angle(off):
        pltpu.async_copy(table.at[i2.at[off]], ga0, sem_a)
        pltpu.async_copy(table.at[i3.at[off]], ga1, sem_a)
        pltpu.async_copy(table.at[i4.at[off]], ga2, sem_a)

    def fire_torsion(off):
        pltpu.async_copy(table.at[i5.at[off]], gt0, sem_t)
        pltpu.async_copy(table.at[i6.at[off]], gt1, sem_t)
        pltpu.async_copy(table.at[i7.at[off]], gt2, sem_t)
        pltpu.async_copy(table.at[i8.at[off]], gt3, sem_t)

    def drain(n, buf, sem):
        for _ in range(n):
            pltpu.make_async_copy(out.at[0, pl.ds(0, CHUNK)], buf, sem).wait()

    pltpu.async_copy(ob, out.at[0, pl.ds(base, CHUNK)], sem_ob)
    pltpu.async_copy(oa, out.at[1, pl.ds(base, CHUNK)], sem_oa)
    pltpu.async_copy(ot, out.at[2, pl.ds(base, CHUNK)], sem_ot)
    fire_bond(pl.ds(0, CHUNK))

    def lo(ref, r, j):
        return ref[r, pl.ds(j * 16, 16)]

    def hi(ref, r, j):
        return ref[r, pl.ds(H + j * 16, 16)]

    def chunk(c, carry):
        off = pl.ds(c * CHUNK, CHUNK)
        nxt = lax.min(c + 1, CHUNKS_PER_TILE - 1)
        off_n = pl.ds(nxt * CHUNK, CHUNK)
        rows = pl.ds(pl.multiple_of(base + c * CHUNK, 8), CHUNK)

        drain(2, gb0, sem_b)
        fire_angle(off)
        drain(1, ob, sem_ob)

        def bond_row(r, cr):
            for j in range(H // 16):
                ob[r, pl.ds(j * 16, 16)] = lo(gb0, r, j) + hi(gb1, r, j)
                ob[r, pl.ds(H + j * 16, 16)] = lo(gb1, r, j) + hi(gb0, r, j)
            return cr
        lax.fori_loop(0, CHUNK, bond_row, 0)
        pltpu.async_copy(ob, out.at[0, rows], sem_ob)

        drain(3, ga0, sem_a)
        fire_torsion(off)
        drain(1, oa, sem_oa)

        def angle_row(r, cr):
            for j in range(H // 16):
                mid = lo(ga2, r, j)
                oa[r, pl.ds(j * 16, 16)] = lo(ga0, r, j) + mid + hi(ga1, r, j)
                oa[r, pl.ds(H + j * 16, 16)] = lo(ga1, r, j) + mid + hi(ga0, r, j)
            return cr
        lax.fori_loop(0, CHUNK, angle_row, 0)
        pltpu.async_copy(oa, out.at[1, rows], sem_oa)

        drain(4, gt0, sem_t)
        fire_bond(off_n)
        drain(1, ot, sem_ot)

        def torsion_row(r, cr):
            for j in range(H // 16):
                ot[r, pl.ds(j * 16, 16)] = (lo(gt0, r, j) + hi(gt1, r, j)
                                            + lo(gt2, r, j) + hi(gt3, r, j))
            return cr
        lax.fori_loop(0, CHUNK, torsion_row, 0)
        pltpu.async_copy(ot, out.at[2, rows], sem_ot)
        return carry

    lax.fori_loop(0, CHUNKS_PER_TILE, chunk, 0)

    drain(2, gb0, sem_b)
    drain(1, ob, sem_ob)
    drain(1, oa, sem_oa)
    drain(1, ot, sem_ot)


_sc_gather = functools.partial(
    pl.kernel,
    out_type=jax.ShapeDtypeStruct((3, NPAD, D), jnp.float32),
    mesh=plsc.VectorSubcoreMesh(core_axis_name="c", subcore_axis_name="s"),
    scratch_types=(
        [pltpu.VMEM((PER_TILE,), jnp.int32)] * NSTREAM
        + [pltpu.VMEM((CHUNK, D), jnp.float32)] * 12
        + [pltpu.SemaphoreType.DMA] * 7
    ),
)(_sc_body)


def _tail_body(hb, ha, ht, br, ar, tr,
               wbr, bb1, bW2, bb2, bW3, bb3,
               war, ab1, aW2, ab2, aW3, ab3,
               wtr, tb1, tW2, tb2, tW3, tb3,
               ob, oa, ot):
    def two_dir(h, r, wr, b1, W2, b2, W3, b3):
        c = r * wr + b1
        z = jnp.maximum(jnp.concatenate([h[0, :, :H] + c, h[0, :, H:] + c], axis=0), 0.0)
        z = jnp.maximum(jnp.dot(z, W2, preferred_element_type=jnp.float32) + b2, 0.0)
        zsum = z[:ROW_BLK] + z[ROW_BLK:]
        return jnp.dot(zsum, W3, preferred_element_type=jnp.float32) + 2.0 * b3

    ob[...] = two_dir(hb[...], br[...], wbr[...], bb1[...],
                      bW2[...], bb2[...], bW3[...], bb3[...])
    oa[...] = two_dir(ha[...], ar[...], war[...], ab1[...],
                      aW2[...], ab2[...], aW3[...], ab3[...])
    z = jnp.maximum(ht[0, :, :H] + tr[...] * wtr[...] + tb1[...], 0.0)
    z = jnp.maximum(jnp.dot(z, tW2[...], preferred_element_type=jnp.float32) + tb2[...], 0.0)
    ot[...] = jnp.dot(z, tW3[...], preferred_element_type=jnp.float32) + tb3[...]


def _tail(h3, br, ar, tr, wts):
    def hspec(s):
        return pl.BlockSpec((1, ROW_BLK, D), lambda i, s=s: (s, i, 0))

    rspec = pl.BlockSpec((ROW_BLK, 1), lambda i: (i, 0))

    def full(a):
        return pl.BlockSpec(a.shape, lambda i: tuple(0 for _ in a.shape))

    return pl.pallas_call(
        _tail_body,
        grid=(NPAD // ROW_BLK,),
        in_specs=[hspec(0), hspec(1), hspec(2)] + [rspec] * 3 + [full(w) for w in wts],
        out_specs=[pl.BlockSpec((ROW_BLK, OUT), lambda i: (i, 0))] * 3,
        out_shape=[jax.ShapeDtypeStruct((NPAD, OUT), jnp.float32)] * 3,
    )(h3, h3, h3, br, ar, tr, *wts)


def kernel(x_atom, bond_idx, angle_idx, torsion_idx,
           bond_repr, angle_repr, torsion_repr,
           bond_W1, bond_b1, bond_W2, bond_b2, bond_W3, bond_b3,
           angle_W1, angle_b1, angle_W2, angle_b2, angle_W3, angle_b3,
           torsion_W1, torsion_b1, torsion_W2, torsion_b2, torsion_W3, torsion_b3):
    wcat = jnp.concatenate(
        [bond_W1[0:D], bond_W1[D:2 * D],
         angle_W1[0:D], angle_W1[2 * D:3 * D],
         angle_W1[D:2 * D], angle_W1[D:2 * D],
         torsion_W1[0:D], torsion_W1[D:2 * D],
         torsion_W1[2 * D:3 * D], torsion_W1[3 * D:4 * D]], axis=1)
    table = _project(x_atom, wcat).reshape(NTAB * N_ATOM, D)

    pad = NPAD - N_EDGE

    def col(a, j, tab):
        return jnp.pad(a[:, j].astype(jnp.int32) + tab * N_ATOM, (0, pad))

    idx = jnp.stack(
        [col(bond_idx, 0, 0), col(bond_idx, 1, 0),
         col(angle_idx, 0, 1), col(angle_idx, 2, 1), col(angle_idx, 1, 2),
         col(torsion_idx, 0, 3), col(torsion_idx, 1, 3),
         col(torsion_idx, 2, 4), col(torsion_idx, 3, 4)]).reshape(NSTREAM * NPAD)
    h3 = _sc_gather(table, idx)

    reprs = [jnp.pad(r, ((0, pad), (0, 0)))
             for r in (bond_repr, angle_repr, torsion_repr)]

    wts = [bond_W1[2 * D].reshape(1, H), bond_b1.reshape(1, H),
           bond_W2, bond_b2.reshape(1, H), bond_W3, bond_b3.reshape(1, OUT),
           angle_W1[3 * D].reshape(1, H), angle_b1.reshape(1, H),
           angle_W2, angle_b2.reshape(1, H), angle_W3, angle_b3.reshape(1, OUT),
           torsion_W1[4 * D].reshape(1, H), torsion_b1.reshape(1, H),
           torsion_W2, torsion_b2.reshape(1, H), torsion_W3, torsion_b3.reshape(1, OUT)]
    return tuple(o[:N_EDGE]
                 for o in _tail(h3, reprs[0], reprs[1], reprs[2], wts))


# speedup vs baseline: 1.0023x; 1.0023x over previous
import functools

import jax
import jax.numpy as jnp
from jax import lax
from jax.experimental import pallas as pl
from jax.experimental.pallas import tpu as pltpu
from jax.experimental.pallas import tpu_sc as plsc

N_ATOM = 50000
N_EDGE = 50000
D = 128
H = 64
OUT = 10
NTAB = 5
NSTREAM = 9

NC = 2
NS = 16
NW = NC * NS
CHUNK = 56
CHUNKS_PER_TILE = 28
PER_TILE = CHUNK * CHUNKS_PER_TILE
NPAD = PER_TILE * NW

ATOM_BLK = 2000
ROW_BLK = 1568


def _proj_body(x_ref, w_ref, out_ref):
    y = jnp.dot(x_ref[...], w_ref[...], preferred_element_type=jnp.float32)
    for k in range(NTAB):
        out_ref[k] = y[:, k * D:(k + 1) * D]


def _project(x_atom, wcat):
    return pl.pallas_call(
        _proj_body,
        grid=(N_ATOM // ATOM_BLK,),
        in_specs=[
            pl.BlockSpec((ATOM_BLK, D), lambda i: (i, 0)),
            pl.BlockSpec((D, NTAB * D), lambda i: (0, 0)),
        ],
        out_specs=pl.BlockSpec((NTAB, ATOM_BLK, D), lambda i: (0, i, 0)),
        out_shape=jax.ShapeDtypeStruct((NTAB, N_ATOM, D), jnp.float32),
    )(x_atom, wcat)


def _sc_body(table, idx, out,
             i0, i1, i2, i3, i4, i5, i6, i7, i8,
             gb0, gb1, ga0, ga1, ga2, gt0, gt1, gt2, gt3,
             ob, oa, ot,
             sem_i, sem_b, sem_a, sem_t, sem_ob, sem_oa, sem_ot):
    ib = (i0, i1, i2, i3, i4, i5, i6, i7, i8)
    wid = lax.axis_index("s") * NC + lax.axis_index("c")
    base = wid * PER_TILE

    cps = [pltpu.async_copy(idx.at[pl.ds(k * NPAD + base, PER_TILE)], ib[k], sem_i)
           for k in range(NSTREAM)]
    for cp in cps:
        cp.wait()

    def fire_bond(off):
        pltpu.async_copy(table.at[i0.at[off]], gb0, sem_b)
        pltpu.async_copy(table.at[i1.at[off]], gb1, sem_b)

    def fire_angle(off):
        pltpu.async_copy(table.at[i2.at[off]], ga0, sem_a)
        pltpu.async_copy(table.at[i3.at[off]], ga1, sem_a)
        pltpu.async_copy(table.at[i4.at[off]], ga2, sem_a)

    def fire_torsion(off):
        pltpu.async_copy(table.at[i5.at[off]], gt0, sem_t)
        pltpu.async_copy(table.at[i6.at[off]], gt1, sem_t)
        pltpu.async_copy(table.at[i7.at[off]], gt2, sem_t)
        pltpu.async_copy(table.at[i8.at[off]], gt3, sem_t)

    def drain(n, buf, sem):
        for _ in range(n):
            pltpu.make_async_copy(out.at[0, pl.ds(0, CHUNK)], buf, sem).wait()

    pltpu.async_copy(ob, out.at[0, pl.ds(base, CHUNK)], sem_ob)
    pltpu.async_copy(oa, out.at[1, pl.ds(base, CHUNK)], sem_oa)
    pltpu.async_copy(ot, out.at[2, pl.ds(base, CHUNK)], sem_ot)
    fire_bond(pl.ds(0, CHUNK))

    def lo(ref, r, j):
        return ref[r, pl.ds(j * 16, 16)]

    def hi(ref, r, j):
        return ref[r, pl.ds(H + j * 16, 16)]

    def chunk(c, carry):
        off = pl.ds(c * CHUNK, CHUNK)
        nxt = lax.min(c + 1, CHUNKS_PER_TILE - 1)
        off_n = pl.ds(nxt * CHUNK, CHUNK)
        rows = pl.ds(pl.multiple_of(base + c * CHUNK, 8), CHUNK)

        drain(2, gb0, sem_b)
        fire_angle(off)
        drain(1, ob, sem_ob)

        def bond_row(r, cr):
            for j in range(H // 16):
                ob[r, pl.ds(j * 16, 16)] = lo(gb0, r, j) + hi(gb1, r, j)
                ob[r, pl.ds(H + j * 16, 16)] = lo(gb1, r, j) + hi(gb0, r, j)
            return cr
        lax.fori_loop(0, CHUNK, bond_row, 0)
        pltpu.async_copy(ob, out.at[0, rows], sem_ob)

        drain(3, ga0, sem_a)
        fire_torsion(off)
        drain(1, oa, sem_oa)

        def angle_row(r, cr):
            for j in range(H // 16):
                mid = lo(ga2, r, j)
                oa[r, pl.ds(j * 16, 16)] = lo(ga0, r, j) + mid + hi(ga1, r, j)
                oa[r, pl.ds(H + j * 16, 16)] = lo(ga1, r, j) + mid + hi(ga0, r, j)
            return cr
        lax.fori_loop(0, CHUNK, angle_row, 0)
        pltpu.async_copy(oa, out.at[1, rows], sem_oa)

        drain(4, gt0, sem_t)
        fire_bond(off_n)
        drain(1, ot, sem_ot)

        def torsion_row(r, cr):
            for j in range(H // 16):
                ot[r, pl.ds(j * 16, 16)] = (lo(gt0, r, j) + hi(gt1, r, j)
                                            + lo(gt2, r, j) + hi(gt3, r, j))
            return cr
        lax.fori_loop(0, CHUNK, torsion_row, 0)
        pltpu.async_copy(ot, out.at[2, rows], sem_ot)
        return carry

    lax.fori_loop(0, CHUNKS_PER_TILE, chunk, 0)

    drain(2, gb0, sem_b)
    drain(1, ob, sem_ob)
    drain(1, oa, sem_oa)
    drain(1, ot, sem_ot)


_sc_gather = functools.partial(
    pl.kernel,
    out_type=jax.ShapeDtypeStruct((3, NPAD, D), jnp.float32),
    mesh=plsc.VectorSubcoreMesh(core_axis_name="c", subcore_axis_name="s"),
    scratch_types=(
        [pltpu.VMEM((PER_TILE,), jnp.int32)] * NSTREAM
        + [pltpu.VMEM((CHUNK, D), jnp.float32)] * 12
        + [pltpu.SemaphoreType.DMA] * 7
    ),
)(_sc_body)


def _tail_body(hb, ha, ht, br, ar, tr,
               wbr, bb1, bW2, bb2, bW3, bb3,
               war, ab1, aW2, ab2, aW3, ab3,
               wtr, tb1, tW2, tb2, tW3, tb3,
               ob, oa, ot):
    def two_dir(h, r, wr, b1, W2, b2, W3, b3):
        c = r * wr + b1
        zf = jnp.maximum(h[0, :, :H] + c, 0.0)
        zr = jnp.maximum(h[0, :, H:] + c, 0.0)
        zf = jnp.maximum(jnp.dot(zf, W2, preferred_element_type=jnp.float32) + b2, 0.0)
        zr = jnp.maximum(jnp.dot(zr, W2, preferred_element_type=jnp.float32) + b2, 0.0)
        return jnp.dot(zf + zr, W3, preferred_element_type=jnp.float32) + 2.0 * b3

    ob[...] = two_dir(hb[...], br[...], wbr[...], bb1[...],
                      bW2[...], bb2[...], bW3[...], bb3[...])
    oa[...] = two_dir(ha[...], ar[...], war[...], ab1[...],
                      aW2[...], ab2[...], aW3[...], ab3[...])
    z = jnp.maximum(ht[0, :, :H] + tr[...] * wtr[...] + tb1[...], 0.0)
    z = jnp.maximum(jnp.dot(z, tW2[...], preferred_element_type=jnp.float32) + tb2[...], 0.0)
    ot[...] = jnp.dot(z, tW3[...], preferred_element_type=jnp.float32) + tb3[...]


def _tail(h3, br, ar, tr, wts):
    def hspec(s):
        return pl.BlockSpec((1, ROW_BLK, D), lambda i, s=s: (s, i, 0))

    rspec = pl.BlockSpec((ROW_BLK, 1), lambda i: (i, 0))

    def full(a):
        return pl.BlockSpec(a.shape, lambda i: tuple(0 for _ in a.shape))

    return pl.pallas_call(
        _tail_body,
        grid=(NPAD // ROW_BLK,),
        in_specs=[hspec(0), hspec(1), hspec(2)] + [rspec] * 3 + [full(w) for w in wts],
        out_specs=[pl.BlockSpec((ROW_BLK, OUT), lambda i: (i, 0))] * 3,
        out_shape=[jax.ShapeDtypeStruct((NPAD, OUT), jnp.float32)] * 3,
    )(h3, h3, h3, br, ar, tr, *wts)


def kernel(x_atom, bond_idx, angle_idx, torsion_idx,
           bond_repr, angle_repr, torsion_repr,
           bond_W1, bond_b1, bond_W2, bond_b2, bond_W3, bond_b3,
           angle_W1, angle_b1, angle_W2, angle_b2, angle_W3, angle_b3,
           torsion_W1, torsion_b1, torsion_W2, torsion_b2, torsion_W3, torsion_b3):
    wcat = jnp.concatenate(
        [bond_W1[0:D], bond_W1[D:2 * D],
         angle_W1[0:D], angle_W1[2 * D:3 * D],
         angle_W1[D:2 * D], angle_W1[D:2 * D],
         torsion_W1[0:D], torsion_W1[D:2 * D],
         torsion_W1[2 * D:3 * D], torsion_W1[3 * D:4 * D]], axis=1)
    table = _project(x_atom, wcat).reshape(NTAB * N_ATOM, D)

    pad = NPAD - N_EDGE

    def col(a, j, tab):
        return jnp.pad(a[:, j].astype(jnp.int32) + tab * N_ATOM, (0, pad))

    idx = jnp.stack(
        [col(bond_idx, 0, 0), col(bond_idx, 1, 0),
         col(angle_idx, 0, 1), col(angle_idx, 2, 1), col(angle_idx, 1, 2),
         col(torsion_idx, 0, 3), col(torsion_idx, 1, 3),
         col(torsion_idx, 2, 4), col(torsion_idx, 3, 4)]).reshape(NSTREAM * NPAD)
    h3 = _sc_gather(table, idx)

    reprs = [jnp.pad(r, ((0, pad), (0, 0)))
             for r in (bond_repr, angle_repr, torsion_repr)]

    wts = [bond_W1[2 * D].reshape(1, H), bond_b1.reshape(1, H),
           bond_W2, bond_b2.reshape(1, H), bond_W3, bond_b3.reshape(1, OUT),
           angle_W1[3 * D].reshape(1, H), angle_b1.reshape(1, H),
           angle_W2, angle_b2.reshape(1, H), angle_W3, angle_b3.reshape(1, OUT),
           torsion_W1[4 * D].reshape(1, H), torsion_b1.reshape(1, H),
           torsion_W2, torsion_b2.reshape(1, H), torsion_W3, torsion_b3.reshape(1, OUT)]
    return tuple(o[:N_EDGE]
                 for o in _tail(h3, reprs[0], reprs[1], reprs[2], wts))


# speedup vs baseline: 1.1165x; 1.1139x over previous
import functools

import jax
import jax.numpy as jnp
from jax import lax
from jax.experimental import pallas as pl
from jax.experimental.pallas import tpu as pltpu
from jax.experimental.pallas import tpu_sc as plsc

N_ATOM = 50000
N_EDGE = 50000
D = 128
H = 64
OUT = 10
NTAB = 5
NSTREAM = 9

NC = 2
NS = 16
NW = NC * NS
CHUNK = 56
CHUNKS_PER_TILE = 28
PER_TILE = CHUNK * CHUNKS_PER_TILE
NPAD = PER_TILE * NW

ATOM_BLK = 1000
ROW_BLK = 1000


def _proj_body(x_ref, w_ref, out_ref):
    y = jnp.dot(x_ref[...], w_ref[...], preferred_element_type=jnp.float32)
    for k in range(NTAB):
        out_ref[k] = y[:, k * D:(k + 1) * D]


def _project(x_atom, wcat):
    return pl.pallas_call(
        _proj_body,
        grid=(N_ATOM // ATOM_BLK,),
        in_specs=[
            pl.BlockSpec((ATOM_BLK, D), lambda i: (i, 0)),
            pl.BlockSpec((D, NTAB * D), lambda i: (0, 0)),
        ],
        out_specs=pl.BlockSpec((NTAB, ATOM_BLK, D), lambda i: (0, i, 0)),
        out_shape=jax.ShapeDtypeStruct((NTAB, N_ATOM, D), jnp.float32),
    )(x_atom, wcat)


def _sc_body(table, idx, out,
             i0, i1, i2, i3, i4, i5, i6, i7, i8,
             gb0, gb1, ga0, ga1, ga2, gt0, gt1, gt2, gt3,
             ob, oa, ot,
             sem_i, sem_b, sem_a, sem_t, sem_ob, sem_oa, sem_ot):
    ib = (i0, i1, i2, i3, i4, i5, i6, i7, i8)
    wid = lax.axis_index("s") * NC + lax.axis_index("c")
    base = wid * PER_TILE

    cps = [pltpu.async_copy(idx.at[pl.ds(k * NPAD + base, PER_TILE)], ib[k], sem_i)
           for k in range(NSTREAM)]
    for cp in cps:
        cp.wait()

    def fire_bond(off):
        pltpu.async_copy(table.at[i0.at[off]], gb0, sem_b)
        pltpu.async_copy(table.at[i1.at[off]], gb1, sem_b)

    def fire_angle(off):
        pltpu.async_copy(table.at[i2.at[off]], ga0, sem_a)
        pltpu.async_copy(table.at[i3.at[off]], ga1, sem_a)
        pltpu.async_copy(table.at[i4.at[off]], ga2, sem_a)

    def fire_torsion(off):
        pltpu.async_copy(table.at[i5.at[off]], gt0, sem_t)
        pltpu.async_copy(table.at[i6.at[off]], gt1, sem_t)
        pltpu.async_copy(table.at[i7.at[off]], gt2, sem_t)
        pltpu.async_copy(table.at[i8.at[off]], gt3, sem_t)

    def drain(n, buf, sem):
        for _ in range(n):
            pltpu.make_async_copy(out.at[0, pl.ds(0, CHUNK)], buf, sem).wait()

    pltpu.async_copy(ob, out.at[0, pl.ds(base, CHUNK)], sem_ob)
    pltpu.async_copy(oa, out.at[1, pl.ds(base, CHUNK)], sem_oa)
    pltpu.async_copy(ot, out.at[2, pl.ds(base, CHUNK)], sem_ot)
    fire_bond(pl.ds(0, CHUNK))

    def lo(ref, r, j):
        return ref[r, pl.ds(j * 16, 16)]

    def hi(ref, r, j):
        return ref[r, pl.ds(H + j * 16, 16)]

    def chunk(c, carry):
        off = pl.ds(c * CHUNK, CHUNK)
        nxt = lax.min(c + 1, CHUNKS_PER_TILE - 1)
        off_n = pl.ds(nxt * CHUNK, CHUNK)
        rows = pl.ds(pl.multiple_of(base + c * CHUNK, 8), CHUNK)

        drain(2, gb0, sem_b)
        fire_angle(off)
        drain(1, ob, sem_ob)

        def bond_row(r, cr):
            for j in range(H // 16):
                ob[r, pl.ds(j * 16, 16)] = lo(gb0, r, j) + hi(gb1, r, j)
                ob[r, pl.ds(H + j * 16, 16)] = lo(gb1, r, j) + hi(gb0, r, j)
            return cr
        lax.fori_loop(0, CHUNK, bond_row, 0)
        pltpu.async_copy(ob, out.at[0, rows], sem_ob)

        drain(3, ga0, sem_a)
        fire_torsion(off)
        drain(1, oa, sem_oa)

        def angle_row(r, cr):
            for j in range(H // 16):
                mid = lo(ga2, r, j)
                oa[r, pl.ds(j * 16, 16)] = lo(ga0, r, j) + mid + hi(ga1, r, j)
                oa[r, pl.ds(H + j * 16, 16)] = lo(ga1, r, j) + mid + hi(ga0, r, j)
            return cr
        lax.fori_loop(0, CHUNK, angle_row, 0)
        pltpu.async_copy(oa, out.at[1, rows], sem_oa)

        drain(4, gt0, sem_t)
        fire_bond(off_n)
        drain(1, ot, sem_ot)

        def torsion_row(r, cr):
            for j in range(H // 16):
                ot[r, pl.ds(j * 16, 16)] = (lo(gt0, r, j) + hi(gt1, r, j)
                                            + lo(gt2, r, j) + hi(gt3, r, j))
            return cr
        lax.fori_loop(0, CHUNK, torsion_row, 0)
        pltpu.async_copy(ot, out.at[2, rows], sem_ot)
        return carry

    lax.fori_loop(0, CHUNKS_PER_TILE, chunk, 0)

    drain(2, gb0, sem_b)
    drain(1, ob, sem_ob)
    drain(1, oa, sem_oa)
    drain(1, ot, sem_ot)


_sc_gather = functools.partial(
    pl.kernel,
    out_type=jax.ShapeDtypeStruct((3, NPAD, D), jnp.float32),
    mesh=plsc.VectorSubcoreMesh(core_axis_name="c", subcore_axis_name="s"),
    scratch_types=(
        [pltpu.VMEM((PER_TILE,), jnp.int32)] * NSTREAM
        + [pltpu.VMEM((CHUNK, D), jnp.float32)] * 12
        + [pltpu.SemaphoreType.DMA] * 7
    ),
)(_sc_body)


def _tail_body(hb, ha, ht, br, ar, tr,
               wbr, bb1, bW2, bb2, bW3, bb3,
               war, ab1, aW2, ab2, aW3, ab3,
               wtr, tb1, tW2, tb2, tW3, tb3,
               ob, oa, ot):
    def two_dir(h, r, wr, b1, W2, b2, W3, b3):
        zf = jnp.maximum(h[0, :, :H] + r * wr + b1, 0.0)
        zr = jnp.maximum(h[0, :, H:] + r * wr + b1, 0.0)
        zf = jnp.maximum(jnp.dot(zf, W2, preferred_element_type=jnp.float32) + b2, 0.0)
        zr = jnp.maximum(jnp.dot(zr, W2, preferred_element_type=jnp.float32) + b2, 0.0)
        return jnp.dot(zf + zr, W3, preferred_element_type=jnp.float32) + 2.0 * b3

    ob[...] = two_dir(hb[...], br[...], wbr[...], bb1[...],
                      bW2[...], bb2[...], bW3[...], bb3[...])
    oa[...] = two_dir(ha[...], ar[...], war[...], ab1[...],
                      aW2[...], ab2[...], aW3[...], ab3[...])
    z = jnp.maximum(ht[0, :, :H] + tr[...] * wtr[...] + tb1[...], 0.0)
    z = jnp.maximum(jnp.dot(z, tW2[...], preferred_element_type=jnp.float32) + tb2[...], 0.0)
    ot[...] = jnp.dot(z, tW3[...], preferred_element_type=jnp.float32) + tb3[...]


def _tail(h3, br, ar, tr, wts):
    def hspec(s):
        return pl.BlockSpec((1, ROW_BLK, D), lambda i, s=s: (s, i, 0))

    rspec = pl.BlockSpec((ROW_BLK, 1), lambda i: (i, 0))

    def full(a):
        return pl.BlockSpec(a.shape, lambda i: tuple(0 for _ in a.shape))

    return pl.pallas_call(
        _tail_body,
        grid=(N_EDGE // ROW_BLK,),
        in_specs=[hspec(0), hspec(1), hspec(2)] + [rspec] * 3 + [full(w) for w in wts],
        out_specs=[pl.BlockSpec((ROW_BLK, OUT), lambda i: (i, 0))] * 3,
        out_shape=[jax.ShapeDtypeStruct((N_EDGE, OUT), jnp.float32)] * 3,
    )(h3, h3, h3, br, ar, tr, *wts)


def kernel(x_atom, bond_idx, angle_idx, torsion_idx,
           bond_repr, angle_repr, torsion_repr,
           bond_W1, bond_b1, bond_W2, bond_b2, bond_W3, bond_b3,
           angle_W1, angle_b1, angle_W2, angle_b2, angle_W3, angle_b3,
           torsion_W1, torsion_b1, torsion_W2, torsion_b2, torsion_W3, torsion_b3):
    wcat = jnp.concatenate(
        [bond_W1[0:D], bond_W1[D:2 * D],
         angle_W1[0:D], angle_W1[2 * D:3 * D],
         angle_W1[D:2 * D], angle_W1[D:2 * D],
         torsion_W1[0:D], torsion_W1[D:2 * D],
         torsion_W1[2 * D:3 * D], torsion_W1[3 * D:4 * D]], axis=1)
    table = _project(x_atom, wcat).reshape(NTAB * N_ATOM, D)

    pad = NPAD - N_EDGE

    def col(a, j, tab):
        return jnp.pad(a[:, j].astype(jnp.int32) + tab * N_ATOM, (0, pad))

    idx = jnp.stack(
        [col(bond_idx, 0, 0), col(bond_idx, 1, 0),
         col(angle_idx, 0, 1), col(angle_idx, 2, 1), col(angle_idx, 1, 2),
         col(torsion_idx, 0, 3), col(torsion_idx, 1, 3),
         col(torsion_idx, 2, 4), col(torsion_idx, 3, 4)]).reshape(NSTREAM * NPAD)
    h3 = _sc_gather(table, idx)

    wts = [bond_W1[2 * D].reshape(1, H), bond_b1.reshape(1, H),
           bond_W2, bond_b2.reshape(1, H), bond_W3, bond_b3.reshape(1, OUT),
           angle_W1[3 * D].reshape(1, H), angle_b1.reshape(1, H),
           angle_W2, angle_b2.reshape(1, H), angle_W3, angle_b3.reshape(1, OUT),
           torsion_W1[4 * D].reshape(1, H), torsion_b1.reshape(1, H),
           torsion_W2, torsion_b2.reshape(1, H), torsion_W3, torsion_b3.reshape(1, OUT)]
    return tuple(_tail(h3, bond_repr, angle_repr, torsion_repr, wts))


# speedup vs baseline: 1.1582x; 1.0374x over previous
import functools

import jax
import jax.numpy as jnp
from jax import lax
from jax.experimental import pallas as pl
from jax.experimental.pallas import tpu as pltpu
from jax.experimental.pallas import tpu_sc as plsc

N_ATOM = 50000
N_EDGE = 50000
D = 128
H = 64
OUT = 10
NTAB = 5
NSTREAM = 9

NC = 2
NS = 16
NW = NC * NS
CHUNK = 56
CHUNKS_PER_TILE = 28
PER_TILE = CHUNK * CHUNKS_PER_TILE
NPAD = PER_TILE * NW

ATOM_BLK = 1000
ROW_BLK = 2000


def _proj_body(x_ref, w_ref, out_ref):
    y = jnp.dot(x_ref[...], w_ref[...], preferred_element_type=jnp.float32)
    for k in range(NTAB):
        out_ref[k] = y[:, k * D:(k + 1) * D]


def _project(x_atom, wcat):
    return pl.pallas_call(
        _proj_body,
        grid=(N_ATOM // ATOM_BLK,),
        in_specs=[
            pl.BlockSpec((ATOM_BLK, D), lambda i: (i, 0)),
            pl.BlockSpec((D, NTAB * D), lambda i: (0, 0)),
        ],
        out_specs=pl.BlockSpec((NTAB, ATOM_BLK, D), lambda i: (0, i, 0)),
        out_shape=jax.ShapeDtypeStruct((NTAB, N_ATOM, D), jnp.float32),
    )(x_atom, wcat)


def _sc_body(table, idx, out,
             i0, i1, i2, i3, i4, i5, i6, i7, i8,
             gb0, gb1, ga0, ga1, ga2, gt0, gt1, gt2, gt3,
             ob, oa, ot,
             sem_i, sem_b, sem_a, sem_t, sem_ob, sem_oa, sem_ot):
    ib = (i0, i1, i2, i3, i4, i5, i6, i7, i8)
    wid = lax.axis_index("s") * NC + lax.axis_index("c")
    base = wid * PER_TILE

    cps = [pltpu.async_copy(idx.at[pl.ds(k * NPAD + base, PER_TILE)], ib[k], sem_i)
           for k in range(NSTREAM)]
    for cp in cps:
        cp.wait()

    def fire_bond(off):
        pltpu.async_copy(table.at[i0.at[off]], gb0, sem_b)
        pltpu.async_copy(table.at[i1.at[off]], gb1, sem_b)

    def fire_angle(off):
        pltpu.async_copy(table.at[i2.at[off]], ga0, sem_a)
        pltpu.async_copy(table.at[i3.at[off]], ga1, sem_a)
        pltpu.async_copy(table.at[i4.at[off]], ga2, sem_a)

    def fire_torsion(off):
        pltpu.async_copy(table.at[i5.at[off]], gt0, sem_t)
        pltpu.async_copy(table.at[i6.at[off]], gt1, sem_t)
        pltpu.async_copy(table.at[i7.at[off]], gt2, sem_t)
        pltpu.async_copy(table.at[i8.at[off]], gt3, sem_t)

    def drain(n, buf, sem):
        for _ in range(n):
            pltpu.make_async_copy(out.at[0, pl.ds(0, CHUNK)], buf, sem).wait()

    pltpu.async_copy(ob, out.at[0, pl.ds(base, CHUNK)], sem_ob)
    pltpu.async_copy(oa, out.at[1, pl.ds(base, CHUNK)], sem_oa)
    pltpu.async_copy(ot, out.at[2, pl.ds(base, CHUNK)], sem_ot)
    fire_bond(pl.ds(0, CHUNK))

    def lo(ref, r, j):
        return ref[r, pl.ds(j * 16, 16)]

    def hi(ref, r, j):
        return ref[r, pl.ds(H + j * 16, 16)]

    def chunk(c, carry):
        off = pl.ds(c * CHUNK, CHUNK)
        nxt = lax.min(c + 1, CHUNKS_PER_TILE - 1)
        off_n = pl.ds(nxt * CHUNK, CHUNK)
        rows = pl.ds(pl.multiple_of(base + c * CHUNK, 8), CHUNK)

        drain(2, gb0, sem_b)
        fire_angle(off)
        drain(1, ob, sem_ob)

        def bond_row(r, cr):
            for j in range(H // 16):
                ob[r, pl.ds(j * 16, 16)] = lo(gb0, r, j) + hi(gb1, r, j)
                ob[r, pl.ds(H + j * 16, 16)] = lo(gb1, r, j) + hi(gb0, r, j)
            return cr
        lax.fori_loop(0, CHUNK, bond_row, 0)
        pltpu.async_copy(ob, out.at[0, rows], sem_ob)

        drain(3, ga0, sem_a)
        fire_torsion(off)
        drain(1, oa, sem_oa)

        def angle_row(r, cr):
            for j in range(H // 16):
                mid = lo(ga2, r, j)
                oa[r, pl.ds(j * 16, 16)] = lo(ga0, r, j) + mid + hi(ga1, r, j)
                oa[r, pl.ds(H + j * 16, 16)] = lo(ga1, r, j) + mid + hi(ga0, r, j)
            return cr
        lax.fori_loop(0, CHUNK, angle_row, 0)
        pltpu.async_copy(oa, out.at[1, rows], sem_oa)

        drain(4, gt0, sem_t)
        fire_bond(off_n)
        drain(1, ot, sem_ot)

        def torsion_row(r, cr):
            for j in range(H // 16):
                ot[r, pl.ds(j * 16, 16)] = (lo(gt0, r, j) + hi(gt1, r, j)
                                            + lo(gt2, r, j) + hi(gt3, r, j))
            return cr
        lax.fori_loop(0, CHUNK, torsion_row, 0)
        pltpu.async_copy(ot, out.at[2, rows], sem_ot)
        return carry

    lax.fori_loop(0, CHUNKS_PER_TILE, chunk, 0)

    drain(2, gb0, sem_b)
    drain(1, ob, sem_ob)
    drain(1, oa, sem_oa)
    drain(1, ot, sem_ot)


_sc_gather = functools.partial(
    pl.kernel,
    out_type=jax.ShapeDtypeStruct((3, NPAD, D), jnp.float32),
    mesh=plsc.VectorSubcoreMesh(core_axis_name="c", subcore_axis_name="s"),
    scratch_types=(
        [pltpu.VMEM((PER_TILE,), jnp.int32)] * NSTREAM
        + [pltpu.VMEM((CHUNK, D), jnp.float32)] * 12
        + [pltpu.SemaphoreType.DMA] * 7
    ),
)(_sc_body)


def _tail_body(hb, ha, ht, br, ar, tr,
               wbr, bb1, bW2, bb2, bW3, bb3,
               war, ab1, aW2, ab2, aW3, ab3,
               wtr, tb1, tW2, tb2, tW3, tb3,
               ob, oa, ot):
    def two_dir(h, r, wr, b1, W2, b2, W3, b3):
        zf = jnp.maximum(h[0, :, :H] + r * wr + b1, 0.0)
        zr = jnp.maximum(h[0, :, H:] + r * wr + b1, 0.0)
        zf = jnp.maximum(jnp.dot(zf, W2, preferred_element_type=jnp.float32) + b2, 0.0)
        zr = jnp.maximum(jnp.dot(zr, W2, preferred_element_type=jnp.float32) + b2, 0.0)
        return jnp.dot(zf + zr, W3, preferred_element_type=jnp.float32) + 2.0 * b3

    ob[...] = two_dir(hb[...], br[...], wbr[...], bb1[...],
                      bW2[...], bb2[...], bW3[...], bb3[...])
    oa[...] = two_dir(ha[...], ar[...], war[...], ab1[...],
                      aW2[...], ab2[...], aW3[...], ab3[...])
    z = jnp.maximum(ht[0, :, :H] + tr[...] * wtr[...] + tb1[...], 0.0)
    z = jnp.maximum(jnp.dot(z, tW2[...], preferred_element_type=jnp.float32) + tb2[...], 0.0)
    ot[...] = jnp.dot(z, tW3[...], preferred_element_type=jnp.float32) + tb3[...]


def _tail(h3, br, ar, tr, wts):
    def hspec(s):
        return pl.BlockSpec((1, ROW_BLK, D), lambda i, s=s: (s, i, 0))

    rspec = pl.BlockSpec((ROW_BLK, 1), lambda i: (i, 0))

    def full(a):
        return pl.BlockSpec(a.shape, lambda i: tuple(0 for _ in a.shape))

    return pl.pallas_call(
        _tail_body,
        grid=(N_EDGE // ROW_BLK,),
        in_specs=[hspec(0), hspec(1), hspec(2)] + [rspec] * 3 + [full(w) for w in wts],
        out_specs=[pl.BlockSpec((ROW_BLK, OUT), lambda i: (i, 0))] * 3,
        out_shape=[jax.ShapeDtypeStruct((N_EDGE, OUT), jnp.float32)] * 3,
    )(h3, h3, h3, br, ar, tr, *wts)


def kernel(x_atom, bond_idx, angle_idx, torsion_idx,
           bond_repr, angle_repr, torsion_repr,
           bond_W1, bond_b1, bond_W2, bond_b2, bond_W3, bond_b3,
           angle_W1, angle_b1, angle_W2, angle_b2, angle_W3, angle_b3,
           torsion_W1, torsion_b1, torsion_W2, torsion_b2, torsion_W3, torsion_b3):
    wcat = jnp.concatenate(
        [bond_W1[0:D], bond_W1[D:2 * D],
         angle_W1[0:D], angle_W1[2 * D:3 * D],
         angle_W1[D:2 * D], angle_W1[D:2 * D],
         torsion_W1[0:D], torsion_W1[D:2 * D],
         torsion_W1[2 * D:3 * D], torsion_W1[3 * D:4 * D]], axis=1)
    table = _project(x_atom, wcat).reshape(NTAB * N_ATOM, D)

    pad = NPAD - N_EDGE

    def col(a, j, tab):
        return jnp.pad(a[:, j].astype(jnp.int32) + tab * N_ATOM, (0, pad))

    idx = jnp.stack(
        [col(bond_idx, 0, 0), col(bond_idx, 1, 0),
         col(angle_idx, 0, 1), col(angle_idx, 2, 1), col(angle_idx, 1, 2),
         col(torsion_idx, 0, 3), col(torsion_idx, 1, 3),
         col(torsion_idx, 2, 4), col(torsion_idx, 3, 4)]).reshape(NSTREAM * NPAD)
    h3 = _sc_gather(table, idx)

    wts = [bond_W1[2 * D].reshape(1, H), bond_b1.reshape(1, H),
           bond_W2, bond_b2.reshape(1, H), bond_W3, bond_b3.reshape(1, OUT),
           angle_W1[3 * D].reshape(1, H), angle_b1.reshape(1, H),
           angle_W2, angle_b2.reshape(1, H), angle_W3, angle_b3.reshape(1, OUT),
           torsion_W1[4 * D].reshape(1, H), torsion_b1.reshape(1, H),
           torsion_W2, torsion_b2.reshape(1, H), torsion_W3, torsion_b3.reshape(1, OUT)]
    return tuple(_tail(h3, bond_repr, angle_repr, torsion_repr, wts))


# speedup vs baseline: 1.1735x; 1.0132x over previous
import functools

import jax
import jax.numpy as jnp
from jax import lax
from jax.experimental import pallas as pl
from jax.experimental.pallas import tpu as pltpu
from jax.experimental.pallas import tpu_sc as plsc

N_ATOM = 50000
N_EDGE = 50000
D = 128
H = 64
OUT = 10
NTAB = 5
NSTREAM = 9

NC = 2
NS = 16
NW = NC * NS
CHUNK = 56
CHUNKS_PER_TILE = 28
PER_TILE = CHUNK * CHUNKS_PER_TILE
NPAD = PER_TILE * NW

ATOM_BLK = 1000
ROW_BLK = 5000


def _proj_body(x_ref, w_ref, out_ref):
    y = jnp.dot(x_ref[...], w_ref[...], preferred_element_type=jnp.float32)
    for k in range(NTAB):
        out_ref[k] = y[:, k * D:(k + 1) * D]


def _project(x_atom, wcat):
    return pl.pallas_call(
        _proj_body,
        grid=(N_ATOM // ATOM_BLK,),
        in_specs=[
            pl.BlockSpec((ATOM_BLK, D), lambda i: (i, 0)),
            pl.BlockSpec((D, NTAB * D), lambda i: (0, 0)),
        ],
        out_specs=pl.BlockSpec((NTAB, ATOM_BLK, D), lambda i: (0, i, 0)),
        out_shape=jax.ShapeDtypeStruct((NTAB, N_ATOM, D), jnp.float32),
    )(x_atom, wcat)


def _sc_body(table, idx, out,
             i0, i1, i2, i3, i4, i5, i6, i7, i8,
             gb0, gb1, ga0, ga1, ga2, gt0, gt1, gt2, gt3,
             ob, oa, ot,
             sem_i, sem_b, sem_a, sem_t, sem_ob, sem_oa, sem_ot):
    ib = (i0, i1, i2, i3, i4, i5, i6, i7, i8)
    wid = lax.axis_index("s") * NC + lax.axis_index("c")
    base = wid * PER_TILE

    cps = [pltpu.async_copy(idx.at[pl.ds(k * NPAD + base, PER_TILE)], ib[k], sem_i)
           for k in range(NSTREAM)]
    for cp in cps:
        cp.wait()

    def fire_bond(off):
        pltpu.async_copy(table.at[i0.at[off]], gb0, sem_b)
        pltpu.async_copy(table.at[i1.at[off]], gb1, sem_b)

    def fire_angle(off):
        pltpu.async_copy(table.at[i2.at[off]], ga0, sem_a)
        pltpu.async_copy(table.at[i3.at[off]], ga1, sem_a)
        pltpu.async_copy(table.at[i4.at[off]], ga2, sem_a)

    def fire_torsion(off):
        pltpu.async_copy(table.at[i5.at[off]], gt0, sem_t)
        pltpu.async_copy(table.at[i6.at[off]], gt1, sem_t)
        pltpu.async_copy(table.at[i7.at[off]], gt2, sem_t)
        pltpu.async_copy(table.at[i8.at[off]], gt3, sem_t)

    def drain(n, buf, sem):
        for _ in range(n):
            pltpu.make_async_copy(out.at[0, pl.ds(0, CHUNK)], buf, sem).wait()

    pltpu.async_copy(ob, out.at[0, pl.ds(base, CHUNK)], sem_ob)
    pltpu.async_copy(oa, out.at[1, pl.ds(base, CHUNK)], sem_oa)
    pltpu.async_copy(ot, out.at[2, pl.ds(base, CHUNK)], sem_ot)
    fire_bond(pl.ds(0, CHUNK))

    def lo(ref, r, j):
        return ref[r, pl.ds(j * 16, 16)]

    def hi(ref, r, j):
        return ref[r, pl.ds(H + j * 16, 16)]

    def chunk(c, carry):
        off = pl.ds(c * CHUNK, CHUNK)
        nxt = lax.min(c + 1, CHUNKS_PER_TILE - 1)
        off_n = pl.ds(nxt * CHUNK, CHUNK)
        rows = pl.ds(pl.multiple_of(base + c * CHUNK, 8), CHUNK)

        drain(2, gb0, sem_b)
        fire_angle(off)
        drain(1, ob, sem_ob)

        def bond_row(r, cr):
            for j in range(H // 16):
                ob[r, pl.ds(j * 16, 16)] = lo(gb0, r, j) + hi(gb1, r, j)
                ob[r, pl.ds(H + j * 16, 16)] = lo(gb1, r, j) + hi(gb0, r, j)
            return cr
        lax.fori_loop(0, CHUNK, bond_row, 0)
        pltpu.async_copy(ob, out.at[0, rows], sem_ob)

        drain(3, ga0, sem_a)
        fire_torsion(off)
        drain(1, oa, sem_oa)

        def angle_row(r, cr):
            for j in range(H // 16):
                mid = lo(ga2, r, j)
                oa[r, pl.ds(j * 16, 16)] = lo(ga0, r, j) + mid + hi(ga1, r, j)
                oa[r, pl.ds(H + j * 16, 16)] = lo(ga1, r, j) + mid + hi(ga0, r, j)
            return cr
        lax.fori_loop(0, CHUNK, angle_row, 0)
        pltpu.async_copy(oa, out.at[1, rows], sem_oa)

        drain(4, gt0, sem_t)
        fire_bond(off_n)
        drain(1, ot, sem_ot)

        def torsion_row(r, cr):
            for j in range(H // 16):
                ot[r, pl.ds(j * 16, 16)] = (lo(gt0, r, j) + hi(gt1, r, j)
                                            + lo(gt2, r, j) + hi(gt3, r, j))
            return cr
        lax.fori_loop(0, CHUNK, torsion_row, 0)
        pltpu.async_copy(ot, out.at[2, rows], sem_ot)
        return carry

    lax.fori_loop(0, CHUNKS_PER_TILE, chunk, 0)

    drain(2, gb0, sem_b)
    drain(1, ob, sem_ob)
    drain(1, oa, sem_oa)
    drain(1, ot, sem_ot)


_sc_gather = functools.partial(
    pl.kernel,
    out_type=jax.ShapeDtypeStruct((3, NPAD, D), jnp.float32),
    mesh=plsc.VectorSubcoreMesh(core_axis_name="c", subcore_axis_name="s"),
    scratch_types=(
        [pltpu.VMEM((PER_TILE,), jnp.int32)] * NSTREAM
        + [pltpu.VMEM((CHUNK, D), jnp.float32)] * 12
        + [pltpu.SemaphoreType.DMA] * 7
    ),
)(_sc_body)


def _tail_body(hb, ha, ht, br, ar, tr,
               wbr, bb1, bW2, bb2, bW3, bb3,
               war, ab1, aW2, ab2, aW3, ab3,
               wtr, tb1, tW2, tb2, tW3, tb3,
               ob, oa, ot):
    def two_dir(h, r, wr, b1, W2, b2, W3, b3):
        zf = jnp.maximum(h[0, :, :H] + r * wr + b1, 0.0)
        zr = jnp.maximum(h[0, :, H:] + r * wr + b1, 0.0)
        zf = jnp.maximum(jnp.dot(zf, W2, preferred_element_type=jnp.float32) + b2, 0.0)
        zr = jnp.maximum(jnp.dot(zr, W2, preferred_element_type=jnp.float32) + b2, 0.0)
        return jnp.dot(zf + zr, W3, preferred_element_type=jnp.float32) + 2.0 * b3

    ob[...] = two_dir(hb[...], br[...], wbr[...], bb1[...],
                      bW2[...], bb2[...], bW3[...], bb3[...])
    oa[...] = two_dir(ha[...], ar[...], war[...], ab1[...],
                      aW2[...], ab2[...], aW3[...], ab3[...])
    z = jnp.maximum(ht[0, :, :H] + tr[...] * wtr[...] + tb1[...], 0.0)
    z = jnp.maximum(jnp.dot(z, tW2[...], preferred_element_type=jnp.float32) + tb2[...], 0.0)
    ot[...] = jnp.dot(z, tW3[...], preferred_element_type=jnp.float32) + tb3[...]


def _tail(h3, br, ar, tr, wts):
    def hspec(s):
        return pl.BlockSpec((1, ROW_BLK, D), lambda i, s=s: (s, i, 0))

    rspec = pl.BlockSpec((ROW_BLK, 1), lambda i: (i, 0))

    def full(a):
        return pl.BlockSpec(a.shape, lambda i: tuple(0 for _ in a.shape))

    return pl.pallas_call(
        _tail_body,
        grid=(N_EDGE // ROW_BLK,),
        in_specs=[hspec(0), hspec(1), hspec(2)] + [rspec] * 3 + [full(w) for w in wts],
        out_specs=[pl.BlockSpec((ROW_BLK, OUT), lambda i: (i, 0))] * 3,
        out_shape=[jax.ShapeDtypeStruct((N_EDGE, OUT), jnp.float32)] * 3,
    )(h3, h3, h3, br, ar, tr, *wts)


def kernel(x_atom, bond_idx, angle_idx, torsion_idx,
           bond_repr, angle_repr, torsion_repr,
           bond_W1, bond_b1, bond_W2, bond_b2, bond_W3, bond_b3,
           angle_W1, angle_b1, angle_W2, angle_b2, angle_W3, angle_b3,
           torsion_W1, torsion_b1, torsion_W2, torsion_b2, torsion_W3, torsion_b3):
    wcat = jnp.concatenate(
        [bond_W1[0:D], bond_W1[D:2 * D],
         angle_W1[0:D], angle_W1[2 * D:3 * D],
         angle_W1[D:2 * D], angle_W1[D:2 * D],
         torsion_W1[0:D], torsion_W1[D:2 * D],
         torsion_W1[2 * D:3 * D], torsion_W1[3 * D:4 * D]], axis=1)
    table = _project(x_atom, wcat).reshape(NTAB * N_ATOM, D)

    pad = NPAD - N_EDGE

    def col(a, j, tab):
        return jnp.pad(a[:, j].astype(jnp.int32) + tab * N_ATOM, (0, pad))

    idx = jnp.stack(
        [col(bond_idx, 0, 0), col(bond_idx, 1, 0),
         col(angle_idx, 0, 1), col(angle_idx, 2, 1), col(angle_idx, 1, 2),
         col(torsion_idx, 0, 3), col(torsion_idx, 1, 3),
         col(torsion_idx, 2, 4), col(torsion_idx, 3, 4)]).reshape(NSTREAM * NPAD)
    h3 = _sc_gather(table, idx)

    wts = [bond_W1[2 * D].reshape(1, H), bond_b1.reshape(1, H),
           bond_W2, bond_b2.reshape(1, H), bond_W3, bond_b3.reshape(1, OUT),
           angle_W1[3 * D].reshape(1, H), angle_b1.reshape(1, H),
           angle_W2, angle_b2.reshape(1, H), angle_W3, angle_b3.reshape(1, OUT),
           torsion_W1[4 * D].reshape(1, H), torsion_b1.reshape(1, H),
           torsion_W2, torsion_b2.reshape(1, H), torsion_W3, torsion_b3.reshape(1, OUT)]
    return tuple(_tail(h3, bond_repr, angle_repr, torsion_repr, wts))


# speedup vs baseline: 1.2103x; 1.0314x over previous
import functools

import jax
import jax.numpy as jnp
from jax import lax
from jax.experimental import pallas as pl
from jax.experimental.pallas import tpu as pltpu
from jax.experimental.pallas import tpu_sc as plsc

N_ATOM = 50000
N_EDGE = 50000
D = 128
H = 64
OUT = 10
NTAB = 5
NSTREAM = 9

NC = 2
NS = 16
NW = NC * NS
CHUNK = 56
CHUNKS_PER_TILE = 28
PER_TILE = CHUNK * CHUNKS_PER_TILE
NPAD = PER_TILE * NW

ATOM_BLK = 2000
ROW_BLK = 5000


def _proj_body(x_ref, w_ref, out_ref):
    y = jnp.dot(x_ref[...], w_ref[...], preferred_element_type=jnp.float32)
    for k in range(NTAB):
        out_ref[k] = y[:, k * D:(k + 1) * D]


def _project(x_atom, wcat):
    return pl.pallas_call(
        _proj_body,
        grid=(N_ATOM // ATOM_BLK,),
        in_specs=[
            pl.BlockSpec((ATOM_BLK, D), lambda i: (i, 0)),
            pl.BlockSpec((D, NTAB * D), lambda i: (0, 0)),
        ],
        out_specs=pl.BlockSpec((NTAB, ATOM_BLK, D), lambda i: (0, i, 0)),
        out_shape=jax.ShapeDtypeStruct((NTAB, N_ATOM, D), jnp.float32),
    )(x_atom, wcat)


def _sc_body(table, idx, out,
             i0, i1, i2, i3, i4, i5, i6, i7, i8,
             gb0, gb1, ga0, ga1, ga2, gt0, gt1, gt2, gt3,
             ob, oa, ot,
             sem_i, sem_b, sem_a, sem_t, sem_ob, sem_oa, sem_ot):
    ib = (i0, i1, i2, i3, i4, i5, i6, i7, i8)
    wid = lax.axis_index("s") * NC + lax.axis_index("c")
    base = wid * PER_TILE

    cps = [pltpu.async_copy(idx.at[pl.ds(k * NPAD + base, PER_TILE)], ib[k], sem_i)
           for k in range(NSTREAM)]
    for cp in cps:
        cp.wait()

    def fire_bond(off):
        pltpu.async_copy(table.at[i0.at[off]], gb0, sem_b)
        pltpu.async_copy(table.at[i1.at[off]], gb1, sem_b)

    def fire_angle(off):
        pltpu.async_copy(table.at[i2.at[off]], ga0, sem_a)
        pltpu.async_copy(table.at[i3.at[off]], ga1, sem_a)
        pltpu.async_copy(table.at[i4.at[off]], ga2, sem_a)

    def fire_torsion(off):
        pltpu.async_copy(table.at[i5.at[off]], gt0, sem_t)
        pltpu.async_copy(table.at[i6.at[off]], gt1, sem_t)
        pltpu.async_copy(table.at[i7.at[off]], gt2, sem_t)
        pltpu.async_copy(table.at[i8.at[off]], gt3, sem_t)

    def drain(n, buf, sem):
        for _ in range(n):
            pltpu.make_async_copy(out.at[0, pl.ds(0, CHUNK)], buf, sem).wait()

    pltpu.async_copy(ob, out.at[0, pl.ds(base, CHUNK)], sem_ob)
    pltpu.async_copy(oa, out.at[1, pl.ds(base, CHUNK)], sem_oa)
    pltpu.async_copy(ot, out.at[2, pl.ds(base, CHUNK)], sem_ot)
    fire_bond(pl.ds(0, CHUNK))

    def lo(ref, r, j):
        return ref[r, pl.ds(j * 16, 16)]

    def hi(ref, r, j):
        return ref[r, pl.ds(H + j * 16, 16)]

    def chunk(c, carry):
        off = pl.ds(c * CHUNK, CHUNK)
        nxt = lax.min(c + 1, CHUNKS_PER_TILE - 1)
        off_n = pl.ds(nxt * CHUNK, CHUNK)
        rows = pl.ds(pl.multiple_of(base + c * CHUNK, 8), CHUNK)

        drain(2, gb0, sem_b)
        fire_angle(off)
        drain(1, ob, sem_ob)

        def bond_row(r, cr):
            for j in range(H // 16):
                ob[r, pl.ds(j * 16, 16)] = lo(gb0, r, j) + hi(gb1, r, j)
                ob[r, pl.ds(H + j * 16, 16)] = lo(gb1, r, j) + hi(gb0, r, j)
            return cr
        lax.fori_loop(0, CHUNK, bond_row, 0)
        pltpu.async_copy(ob, out.at[0, rows], sem_ob)

        drain(3, ga0, sem_a)
        fire_torsion(off)
        drain(1, oa, sem_oa)

        def angle_row(r, cr):
            for j in range(H // 16):
                mid = lo(ga2, r, j)
                oa[r, pl.ds(j * 16, 16)] = lo(ga0, r, j) + mid + hi(ga1, r, j)
                oa[r, pl.ds(H + j * 16, 16)] = lo(ga1, r, j) + mid + hi(ga0, r, j)
            return cr
        lax.fori_loop(0, CHUNK, angle_row, 0)
        pltpu.async_copy(oa, out.at[1, rows], sem_oa)

        drain(4, gt0, sem_t)
        fire_bond(off_n)
        drain(1, ot, sem_ot)

        def torsion_row(r, cr):
            for j in range(H // 16):
                ot[r, pl.ds(j * 16, 16)] = (lo(gt0, r, j) + hi(gt1, r, j)
                                            + lo(gt2, r, j) + hi(gt3, r, j))
            return cr
        lax.fori_loop(0, CHUNK, torsion_row, 0)
        pltpu.async_copy(ot, out.at[2, rows], sem_ot)
        return carry

    lax.fori_loop(0, CHUNKS_PER_TILE, chunk, 0)

    drain(2, gb0, sem_b)
    drain(1, ob, sem_ob)
    drain(1, oa, sem_oa)
    drain(1, ot, sem_ot)


_sc_gather = functools.partial(
    pl.kernel,
    out_type=jax.ShapeDtypeStruct((3, NPAD, D), jnp.float32),
    mesh=plsc.VectorSubcoreMesh(core_axis_name="c", subcore_axis_name="s"),
    scratch_types=(
        [pltpu.VMEM((PER_TILE,), jnp.int32)] * NSTREAM
        + [pltpu.VMEM((CHUNK, D), jnp.float32)] * 12
        + [pltpu.SemaphoreType.DMA] * 7
    ),
)(_sc_body)


def _tail_body(hb, ha, ht, br, ar, tr,
               wbr, bb1, bW2, bb2, bW3, bb3,
               war, ab1, aW2, ab2, aW3, ab3,
               wtr, tb1, tW2, tb2, tW3, tb3,
               ob, oa, ot):
    def two_dir(h, r, wr, b1, W2, b2, W3, b3):
        zf = jnp.maximum(h[0, :, :H] + r * wr + b1, 0.0)
        zr = jnp.maximum(h[0, :, H:] + r * wr + b1, 0.0)
        zf = jnp.maximum(jnp.dot(zf, W2, preferred_element_type=jnp.float32) + b2, 0.0)
        zr = jnp.maximum(jnp.dot(zr, W2, preferred_element_type=jnp.float32) + b2, 0.0)
        return jnp.dot(zf + zr, W3, preferred_element_type=jnp.float32) + 2.0 * b3

    ob[...] = two_dir(hb[...], br[...], wbr[...], bb1[...],
                      bW2[...], bb2[...], bW3[...], bb3[...])
    oa[...] = two_dir(ha[...], ar[...], war[...], ab1[...],
                      aW2[...], ab2[...], aW3[...], ab3[...])
    z = jnp.maximum(ht[0, :, :H] + tr[...] * wtr[...] + tb1[...], 0.0)
    z = jnp.maximum(jnp.dot(z, tW2[...], preferred_element_type=jnp.float32) + tb2[...], 0.0)
    ot[...] = jnp.dot(z, tW3[...], preferred_element_type=jnp.float32) + tb3[...]


def _tail(h3, br, ar, tr, wts):
    def hspec(s):
        return pl.BlockSpec((1, ROW_BLK, D), lambda i, s=s: (s, i, 0))

    rspec = pl.BlockSpec((ROW_BLK, 1), lambda i: (i, 0))

    def full(a):
        return pl.BlockSpec(a.shape, lambda i: tuple(0 for _ in a.shape))

    return pl.pallas_call(
        _tail_body,
        grid=(N_EDGE // ROW_BLK,),
        in_specs=[hspec(0), hspec(1), hspec(2)] + [rspec] * 3 + [full(w) for w in wts],
        out_specs=[pl.BlockSpec((ROW_BLK, OUT), lambda i: (i, 0))] * 3,
        out_shape=[jax.ShapeDtypeStruct((N_EDGE, OUT), jnp.float32)] * 3,
    )(h3, h3, h3, br, ar, tr, *wts)


def kernel(x_atom, bond_idx, angle_idx, torsion_idx,
           bond_repr, angle_repr, torsion_repr,
           bond_W1, bond_b1, bond_W2, bond_b2, bond_W3, bond_b3,
           angle_W1, angle_b1, angle_W2, angle_b2, angle_W3, angle_b3,
           torsion_W1, torsion_b1, torsion_W2, torsion_b2, torsion_W3, torsion_b3):
    wcat = jnp.concatenate(
        [bond_W1[0:D], bond_W1[D:2 * D],
         angle_W1[0:D], angle_W1[2 * D:3 * D],
         angle_W1[D:2 * D], angle_W1[D:2 * D],
         torsion_W1[0:D], torsion_W1[D:2 * D],
         torsion_W1[2 * D:3 * D], torsion_W1[3 * D:4 * D]], axis=1)
    table = _project(x_atom, wcat).reshape(NTAB * N_ATOM, D)

    pad = NPAD - N_EDGE

    def col(a, j, tab):
        return jnp.pad(a[:, j].astype(jnp.int32) + tab * N_ATOM, (0, pad))

    idx = jnp.stack(
        [col(bond_idx, 0, 0), col(bond_idx, 1, 0),
         col(angle_idx, 0, 1), col(angle_idx, 2, 1), col(angle_idx, 1, 2),
         col(torsion_idx, 0, 3), col(torsion_idx, 1, 3),
         col(torsion_idx, 2, 4), col(torsion_idx, 3, 4)]).reshape(NSTREAM * NPAD)
    h3 = _sc_gather(table, idx)

    wts = [bond_W1[2 * D].reshape(1, H), bond_b1.reshape(1, H),
           bond_W2, bond_b2.reshape(1, H), bond_W3, bond_b3.reshape(1, OUT),
           angle_W1[3 * D].reshape(1, H), angle_b1.reshape(1, H),
           angle_W2, angle_b2.reshape(1, H), angle_W3, angle_b3.reshape(1, OUT),
           torsion_W1[4 * D].reshape(1, H), torsion_b1.reshape(1, H),
           torsion_W2, torsion_b2.reshape(1, H), torsion_W3, torsion_b3.reshape(1, OUT)]
    return tuple(_tail(h3, bond_repr, angle_repr, torsion_repr, wts))


# speedup vs baseline: 1.2201x; 1.0081x over previous
import functools

import jax
import jax.numpy as jnp
from jax import lax
from jax.experimental import pallas as pl
from jax.experimental.pallas import tpu as pltpu
from jax.experimental.pallas import tpu_sc as plsc

N_ATOM = 50000
N_EDGE = 50000
D = 128
H = 64
OUT = 10
NTAB = 5
NSTREAM = 9

NC = 2
NS = 16
NW = NC * NS
CHUNK = 56
CHUNKS_PER_TILE = 28
PER_TILE = CHUNK * CHUNKS_PER_TILE
NPAD = PER_TILE * NW

ATOM_BLK = 5000
ROW_BLK = 5000


def _proj_body(x_ref, w_ref, out_ref):
    y = jnp.dot(x_ref[...], w_ref[...], preferred_element_type=jnp.float32)
    for k in range(NTAB):
        out_ref[k] = y[:, k * D:(k + 1) * D]


def _project(x_atom, wcat):
    return pl.pallas_call(
        _proj_body,
        grid=(N_ATOM // ATOM_BLK,),
        in_specs=[
            pl.BlockSpec((ATOM_BLK, D), lambda i: (i, 0)),
            pl.BlockSpec((D, NTAB * D), lambda i: (0, 0)),
        ],
        out_specs=pl.BlockSpec((NTAB, ATOM_BLK, D), lambda i: (0, i, 0)),
        out_shape=jax.ShapeDtypeStruct((NTAB, N_ATOM, D), jnp.float32),
    )(x_atom, wcat)


def _sc_body(table, idx, out,
             i0, i1, i2, i3, i4, i5, i6, i7, i8,
             gb0, gb1, ga0, ga1, ga2, gt0, gt1, gt2, gt3,
             ob, oa, ot,
             sem_i, sem_b, sem_a, sem_t, sem_ob, sem_oa, sem_ot):
    ib = (i0, i1, i2, i3, i4, i5, i6, i7, i8)
    wid = lax.axis_index("s") * NC + lax.axis_index("c")
    base = wid * PER_TILE

    cps = [pltpu.async_copy(idx.at[pl.ds(k * NPAD + base, PER_TILE)], ib[k], sem_i)
           for k in range(NSTREAM)]
    for cp in cps:
        cp.wait()

    def fire_bond(off):
        pltpu.async_copy(table.at[i0.at[off]], gb0, sem_b)
        pltpu.async_copy(table.at[i1.at[off]], gb1, sem_b)

    def fire_angle(off):
        pltpu.async_copy(table.at[i2.at[off]], ga0, sem_a)
        pltpu.async_copy(table.at[i3.at[off]], ga1, sem_a)
        pltpu.async_copy(table.at[i4.at[off]], ga2, sem_a)

    def fire_torsion(off):
        pltpu.async_copy(table.at[i5.at[off]], gt0, sem_t)
        pltpu.async_copy(table.at[i6.at[off]], gt1, sem_t)
        pltpu.async_copy(table.at[i7.at[off]], gt2, sem_t)
        pltpu.async_copy(table.at[i8.at[off]], gt3, sem_t)

    def drain(n, buf, sem):
        for _ in range(n):
            pltpu.make_async_copy(out.at[0, pl.ds(0, CHUNK)], buf, sem).wait()

    pltpu.async_copy(ob, out.at[0, pl.ds(base, CHUNK)], sem_ob)
    pltpu.async_copy(oa, out.at[1, pl.ds(base, CHUNK)], sem_oa)
    pltpu.async_copy(ot, out.at[2, pl.ds(base, CHUNK)], sem_ot)
    fire_bond(pl.ds(0, CHUNK))

    def lo(ref, r, j):
        return ref[r, pl.ds(j * 16, 16)]

    def hi(ref, r, j):
        return ref[r, pl.ds(H + j * 16, 16)]

    def chunk(c, carry):
        off = pl.ds(c * CHUNK, CHUNK)
        nxt = lax.min(c + 1, CHUNKS_PER_TILE - 1)
        off_n = pl.ds(nxt * CHUNK, CHUNK)
        rows = pl.ds(pl.multiple_of(base + c * CHUNK, 8), CHUNK)

        drain(2, gb0, sem_b)
        fire_angle(off)
        drain(1, ob, sem_ob)

        def bond_row(r, cr):
            for j in range(H // 16):
                ob[r, pl.ds(j * 16, 16)] = lo(gb0, r, j) + hi(gb1, r, j)
                ob[r, pl.ds(H + j * 16, 16)] = lo(gb1, r, j) + hi(gb0, r, j)
            return cr
        lax.fori_loop(0, CHUNK, bond_row, 0)
        pltpu.async_copy(ob, out.at[0, rows], sem_ob)

        drain(3, ga0, sem_a)
        fire_torsion(off)
        drain(1, oa, sem_oa)

        def angle_row(r, cr):
            for j in range(H // 16):
                mid = lo(ga2, r, j)
                oa[r, pl.ds(j * 16, 16)] = lo(ga0, r, j) + mid + hi(ga1, r, j)
                oa[r, pl.ds(H + j * 16, 16)] = lo(ga1, r, j) + mid + hi(ga0, r, j)
            return cr
        lax.fori_loop(0, CHUNK, angle_row, 0)
        pltpu.async_copy(oa, out.at[1, rows], sem_oa)

        drain(4, gt0, sem_t)
        fire_bond(off_n)
        drain(1, ot, sem_ot)

        def torsion_row(r, cr):
            for j in range(H // 16):
                ot[r, pl.ds(j * 16, 16)] = (lo(gt0, r, j) + hi(gt1, r, j)
                                            + lo(gt2, r, j) + hi(gt3, r, j))
            return cr
        lax.fori_loop(0, CHUNK, torsion_row, 0)
        pltpu.async_copy(ot, out.at[2, rows], sem_ot)
        return carry

    lax.fori_loop(0, CHUNKS_PER_TILE, chunk, 0)

    drain(2, gb0, sem_b)
    drain(1, ob, sem_ob)
    drain(1, oa, sem_oa)
    drain(1, ot, sem_ot)


_sc_gather = functools.partial(
    pl.kernel,
    out_type=jax.ShapeDtypeStruct((3, NPAD, D), jnp.float32),
    mesh=plsc.VectorSubcoreMesh(core_axis_name="c", subcore_axis_name="s"),
    scratch_types=(
        [pltpu.VMEM((PER_TILE,), jnp.int32)] * NSTREAM
        + [pltpu.VMEM((CHUNK, D), jnp.float32)] * 12
        + [pltpu.SemaphoreType.DMA] * 7
    ),
)(_sc_body)


def _tail_body(hb, ha, ht, br, ar, tr,
               wbr, bb1, bW2, bb2, bW3, bb3,
               war, ab1, aW2, ab2, aW3, ab3,
               wtr, tb1, tW2, tb2, tW3, tb3,
               ob, oa, ot):
    def two_dir(h, r, wr, b1, W2, b2, W3, b3):
        zf = jnp.maximum(h[0, :, :H] + r * wr + b1, 0.0)
        zr = jnp.maximum(h[0, :, H:] + r * wr + b1, 0.0)
        zf = jnp.maximum(jnp.dot(zf, W2, preferred_element_type=jnp.float32) + b2, 0.0)
        zr = jnp.maximum(jnp.dot(zr, W2, preferred_element_type=jnp.float32) + b2, 0.0)
        return jnp.dot(zf + zr, W3, preferred_element_type=jnp.float32) + 2.0 * b3

    ob[...] = two_dir(hb[...], br[...], wbr[...], bb1[...],
                      bW2[...], bb2[...], bW3[...], bb3[...])
    oa[...] = two_dir(ha[...], ar[...], war[...], ab1[...],
                      aW2[...], ab2[...], aW3[...], ab3[...])
    z = jnp.maximum(ht[0, :, :H] + tr[...] * wtr[...] + tb1[...], 0.0)
    z = jnp.maximum(jnp.dot(z, tW2[...], preferred_element_type=jnp.float32) + tb2[...], 0.0)
    ot[...] = jnp.dot(z, tW3[...], preferred_element_type=jnp.float32) + tb3[...]


def _tail(h3, br, ar, tr, wts):
    def hspec(s):
        return pl.BlockSpec((1, ROW_BLK, D), lambda i, s=s: (s, i, 0))

    rspec = pl.BlockSpec((ROW_BLK, 1), lambda i: (i, 0))

    def full(a):
        return pl.BlockSpec(a.shape, lambda i: tuple(0 for _ in a.shape))

    return pl.pallas_call(
        _tail_body,
        grid=(N_EDGE // ROW_BLK,),
        in_specs=[hspec(0), hspec(1), hspec(2)] + [rspec] * 3 + [full(w) for w in wts],
        out_specs=[pl.BlockSpec((ROW_BLK, OUT), lambda i: (i, 0))] * 3,
        out_shape=[jax.ShapeDtypeStruct((N_EDGE, OUT), jnp.float32)] * 3,
    )(h3, h3, h3, br, ar, tr, *wts)


def kernel(x_atom, bond_idx, angle_idx, torsion_idx,
           bond_repr, angle_repr, torsion_repr,
           bond_W1, bond_b1, bond_W2, bond_b2, bond_W3, bond_b3,
           angle_W1, angle_b1, angle_W2, angle_b2, angle_W3, angle_b3,
           torsion_W1, torsion_b1, torsion_W2, torsion_b2, torsion_W3, torsion_b3):
    wcat = jnp.concatenate(
        [bond_W1[0:D], bond_W1[D:2 * D],
         angle_W1[0:D], angle_W1[2 * D:3 * D],
         angle_W1[D:2 * D], angle_W1[D:2 * D],
         torsion_W1[0:D], torsion_W1[D:2 * D],
         torsion_W1[2 * D:3 * D], torsion_W1[3 * D:4 * D]], axis=1)
    table = _project(x_atom, wcat).reshape(NTAB * N_ATOM, D)

    pad = NPAD - N_EDGE

    def col(a, j, tab):
        return jnp.pad(a[:, j].astype(jnp.int32) + tab * N_ATOM, (0, pad))

    idx = jnp.stack(
        [col(bond_idx, 0, 0), col(bond_idx, 1, 0),
         col(angle_idx, 0, 1), col(angle_idx, 2, 1), col(angle_idx, 1, 2),
         col(torsion_idx, 0, 3), col(torsion_idx, 1, 3),
         col(torsion_idx, 2, 4), col(torsion_idx, 3, 4)]).reshape(NSTREAM * NPAD)
    h3 = _sc_gather(table, idx)

    wts = [bond_W1[2 * D].reshape(1, H), bond_b1.reshape(1, H),
           bond_W2, bond_b2.reshape(1, H), bond_W3, bond_b3.reshape(1, OUT),
           angle_W1[3 * D].reshape(1, H), angle_b1.reshape(1, H),
           angle_W2, angle_b2.reshape(1, H), angle_W3, angle_b3.reshape(1, OUT),
           torsion_W1[4 * D].reshape(1, H), torsion_b1.reshape(1, H),
           torsion_W2, torsion_b2.reshape(1, H), torsion_W3, torsion_b3.reshape(1, OUT)]
    return tuple(_tail(h3, bond_repr, angle_repr, torsion_repr, wts))
